# Initial kernel scaffold; baseline (speedup 1.0000x reference)
import jax
import jax.numpy as jnp
from jax.experimental import pallas as pl


def kernel(x, c1w, c1s, c1t, c2w, c2s, c2t, l0_wih, l0_whh, l0_b, l1_wih, l1_whh, l1_b, l2_wih, l2_whh, l2_b, l3_wih, l3_whh, l3_b, head_w, head_b):
    raise NotImplementedError("write your pallas kernel here")


# speedup vs baseline: 47.7119x; 47.7119x over previous
import jax
import jax.numpy as jnp
from jax.experimental import pallas as pl
from jax.experimental.pallas import tpu as pltpu

_H = 64
_NC = 8


def _cell(g, c_prev):
    s = jax.nn.sigmoid(g[:, : 3 * _H])
    gg = jnp.tanh(g[:, 3 * _H:])
    c = s[:, _H:2 * _H] * c_prev + s[:, : _H] * gg
    return s[:, 2 * _H:] * jnp.tanh(c), c


def _conv_bn_relu(x2, bk, w_ref, s_ref, t_ref):
    n = x2.shape[0]
    row = jax.lax.broadcasted_iota(jnp.int32, x2.shape, 0)
    xm = jnp.where(row >= bk, pltpu.roll(x2, bk, 0), 0.0)
    xp = jnp.where(row < n - bk, pltpu.roll(x2, n - bk, 0), 0.0)
    acc = jnp.dot(xm, w_ref[0], preferred_element_type=jnp.float32)
    acc = acc + jnp.dot(x2, w_ref[1], preferred_element_type=jnp.float32)
    acc = acc + jnp.dot(xp, w_ref[2], preferred_element_type=jnp.float32)
    return jnp.maximum(acc * s_ref[...] + t_ref[...], 0.0)


def _fused_kernel(
    x_ref,
    c1w, c1s, c1t,
    c2w, c2s, c2t,
    l0_wih, l1_wih,
    whh_c,
    l0_b, l1_b,
    l2_wih, l2_whh, l2_b,
    l3_wih, l3_b,
    head_w, head_b,
    o_ref,
    pf_ref, pr_ref,
    hf_ref, hr_ref,
):
    T, Bk, Cin = x_ref.shape
    n = T * Bk
    zero = jnp.zeros((Bk, _H), jnp.float32)

    x2 = x_ref[...].reshape(n, Cin)
    h1 = _conv_bn_relu(x2, Bk, c1w, c1s, c1t)
    h2 = _conv_bn_relu(h1, Bk, c2w, c2s, c2t)

    pf_ref[...] = (jnp.dot(h2, l0_wih[...], preferred_element_type=jnp.float32)
                   + l0_b[...]).reshape(T, Bk, 4 * _H)
    pr_ref[...] = (jnp.dot(h2, l1_wih[...], preferred_element_type=jnp.float32)
                   + l1_b[...]).reshape(T, Bk, 4 * _H)

    def step1(i, carry):
        hc, cf, cr = carry
        tr = T - 1 - i
        hh = jnp.dot(hc, whh_c[...], preferred_element_type=jnp.float32)
        hf, cf = _cell(pf_ref[i] + hh[:, : 4 * _H], cf)
        hr, cr = _cell(pr_ref[tr] + hh[:, 4 * _H:], cr)
        hf_ref[i] = hf
        hr_ref[tr] = hr
        return jnp.concatenate([hf, hr], axis=1), cf, cr

    jax.lax.fori_loop(
        0, T, step1, (jnp.zeros((Bk, 2 * _H), jnp.float32), zero, zero))

    pf_ref[...] = (jnp.dot(hf_ref[...].reshape(n, _H), l2_wih[: _H],
                           preferred_element_type=jnp.float32)
                   + jnp.dot(hr_ref[...].reshape(n, _H), l2_wih[_H:],
                             preferred_element_type=jnp.float32)
                   + l2_b[...]).reshape(T, Bk, 4 * _H)

    def step2(i, carry):
        h, c = carry
        g = pf_ref[i] + jnp.dot(h, l2_whh[...],
                                preferred_element_type=jnp.float32)
        return _cell(g, c)

    h2f, _ = jax.lax.fori_loop(0, T, step2, (zero, zero))

    g_rev = (jnp.dot(hf_ref[T - 1], l3_wih[: _H],
                     preferred_element_type=jnp.float32)
             + jnp.dot(hr_ref[T - 1], l3_wih[_H:],
                       preferred_element_type=jnp.float32)
             + l3_b[...])
    h2r, _ = _cell(g_rev, zero)

    o_ref[...] = (jnp.dot(jnp.maximum(h2f, 0.0), head_w[: _H],
                          preferred_element_type=jnp.float32)
                  + jnp.dot(jnp.maximum(h2r, 0.0), head_w[_H:],
                            preferred_element_type=jnp.float32)
                  + head_b[...])


def kernel(x, c1w, c1s, c1t, c2w, c2s, c2t,
           l0_wih, l0_whh, l0_b, l1_wih, l1_whh, l1_b,
           l2_wih, l2_whh, l2_b, l3_wih, l3_whh, l3_b,
           head_w, head_b):
    B, Cin, T = x.shape
    xt = jnp.transpose(x, (2, 0, 1))

    Bk = 64
    while B % Bk:
        Bk //= 2

    z = jnp.zeros((_H, 4 * _H), jnp.float32)
    whh_c = jnp.block([[l0_whh, z], [z, l1_whh]])

    full = lambda *shape: pl.BlockSpec(shape, lambda b: (0,) * len(shape))
    out = pl.pallas_call(
        _fused_kernel,
        out_shape=jax.ShapeDtypeStruct((B, _NC), jnp.float32),
        grid=(B // Bk,),
        in_specs=[
            pl.BlockSpec((T, Bk, Cin), lambda b: (0, b, 0)),
            full(3, Cin, 64), full(1, 64), full(1, 64),
            full(3, 64, 128), full(1, 128), full(1, 128),
            full(2 * _H, 4 * _H), full(2 * _H, 4 * _H),
            full(2 * _H, 8 * _H),
            full(1, 4 * _H), full(1, 4 * _H),
            full(2 * _H, 4 * _H), full(_H, 4 * _H), full(1, 4 * _H),
            full(2 * _H, 4 * _H), full(1, 4 * _H),
            full(2 * _H, _NC), full(1, _NC),
        ],
        out_specs=pl.BlockSpec((Bk, _NC), lambda b: (b, 0)),
        scratch_shapes=[
            pltpu.VMEM((T, Bk, 4 * _H), jnp.float32),
            pltpu.VMEM((T, Bk, 4 * _H), jnp.float32),
            pltpu.VMEM((T, Bk, _H), jnp.float32),
            pltpu.VMEM((T, Bk, _H), jnp.float32),
        ],
        compiler_params=pltpu.CompilerParams(
            dimension_semantics=("parallel",)),
    )(
        xt, c1w, c1s, c1t, c2w, c2s, c2t,
        l0_wih, l1_wih, whh_c, l0_b, l1_b,
        l2_wih, l2_whh, l2_b, l3_wih, l3_b,
        head_w, head_b,
    )
    return out


# speedup vs baseline: 55.1367x; 1.1556x over previous
import jax
import jax.numpy as jnp
from jax.experimental import pallas as pl
from jax.experimental.pallas import tpu as pltpu

_H = 64
_NC = 8


def _cell(g, c_prev):
    s = jax.nn.sigmoid(g[:, : 3 * _H])
    gg = jnp.tanh(g[:, 3 * _H:])
    c = s[:, _H:2 * _H] * c_prev + s[:, : _H] * gg
    return s[:, 2 * _H:] * jnp.tanh(c), c


def _conv_bn_relu(x2, bk, w_ref, s_ref, t_ref):
    n = x2.shape[0]
    row = jax.lax.broadcasted_iota(jnp.int32, x2.shape, 0)
    xm = jnp.where(row >= bk, pltpu.roll(x2, bk, 0), 0.0)
    xp = jnp.where(row < n - bk, pltpu.roll(x2, n - bk, 0), 0.0)
    acc = jnp.dot(xm, w_ref[0], preferred_element_type=jnp.float32)
    acc = acc + jnp.dot(x2, w_ref[1], preferred_element_type=jnp.float32)
    acc = acc + jnp.dot(xp, w_ref[2], preferred_element_type=jnp.float32)
    return jnp.maximum(acc * s_ref[...] + t_ref[...], 0.0)


def _fused_kernel(
    x_ref,
    c1w, c1s, c1t,
    c2w, c2s, c2t,
    l0_wih, l1_wih,
    l0_whh, l1_whh,
    l0_b, l1_b,
    l2_wih, l2_whh, l2_b,
    l3_wih, l3_b,
    head_w, head_b,
    o_ref,
    pf_ref, pr_ref,
    hf_ref, hr_ref,
):
    T, Bk, Cin = x_ref.shape
    n = T * Bk
    zero = jnp.zeros((Bk, _H), jnp.float32)

    x2 = x_ref[...].reshape(n, Cin)
    h1 = _conv_bn_relu(x2, Bk, c1w, c1s, c1t)
    h2 = _conv_bn_relu(h1, Bk, c2w, c2s, c2t)

    pf_ref[...] = (jnp.dot(h2, l0_wih[...], preferred_element_type=jnp.float32)
                   + l0_b[...]).reshape(T, Bk, 4 * _H)
    pr_ref[...] = (jnp.dot(h2, l1_wih[...], preferred_element_type=jnp.float32)
                   + l1_b[...]).reshape(T, Bk, 4 * _H)

    def step1(i, carry):
        hf, cf, hr, cr = carry
        tr = T - 1 - i
        gf = pf_ref[i] + jnp.dot(hf, l0_whh[...],
                                 preferred_element_type=jnp.float32)
        hf, cf = _cell(gf, cf)
        gr = pr_ref[tr] + jnp.dot(hr, l1_whh[...],
                                  preferred_element_type=jnp.float32)
        hr, cr = _cell(gr, cr)
        hf_ref[i] = hf
        hr_ref[tr] = hr
        return hf, cf, hr, cr

    jax.lax.fori_loop(0, T, step1, (zero, zero, zero, zero), unroll=8)

    pf_ref[...] = (jnp.dot(hf_ref[...].reshape(n, _H), l2_wih[: _H],
                           preferred_element_type=jnp.float32)
                   + jnp.dot(hr_ref[...].reshape(n, _H), l2_wih[_H:],
                             preferred_element_type=jnp.float32)
                   + l2_b[...]).reshape(T, Bk, 4 * _H)

    def step2(i, carry):
        h, c = carry
        g = pf_ref[i] + jnp.dot(h, l2_whh[...],
                                preferred_element_type=jnp.float32)
        return _cell(g, c)

    h2f, _ = jax.lax.fori_loop(0, T, step2, (zero, zero), unroll=8)

    g_rev = (jnp.dot(hf_ref[T - 1], l3_wih[: _H],
                     preferred_element_type=jnp.float32)
             + jnp.dot(hr_ref[T - 1], l3_wih[_H:],
                       preferred_element_type=jnp.float32)
             + l3_b[...])
    h2r, _ = _cell(g_rev, zero)

    o_ref[...] = (jnp.dot(jnp.maximum(h2f, 0.0), head_w[: _H],
                          preferred_element_type=jnp.float32)
                  + jnp.dot(jnp.maximum(h2r, 0.0), head_w[_H:],
                            preferred_element_type=jnp.float32)
                  + head_b[...])


def kernel(x, c1w, c1s, c1t, c2w, c2s, c2t,
           l0_wih, l0_whh, l0_b, l1_wih, l1_whh, l1_b,
           l2_wih, l2_whh, l2_b, l3_wih, l3_whh, l3_b,
           head_w, head_b):
    B, Cin, T = x.shape
    xt = jnp.transpose(x, (2, 0, 1))

    Bk = 64
    while B % Bk:
        Bk //= 2

    full = lambda *shape: pl.BlockSpec(shape, lambda b: (0,) * len(shape))
    out = pl.pallas_call(
        _fused_kernel,
        out_shape=jax.ShapeDtypeStruct((B, _NC), jnp.float32),
        grid=(B // Bk,),
        in_specs=[
            pl.BlockSpec((T, Bk, Cin), lambda b: (0, b, 0)),
            full(3, Cin, 64), full(1, 64), full(1, 64),
            full(3, 64, 128), full(1, 128), full(1, 128),
            full(2 * _H, 4 * _H), full(2 * _H, 4 * _H),
            full(_H, 4 * _H), full(_H, 4 * _H),
            full(1, 4 * _H), full(1, 4 * _H),
            full(2 * _H, 4 * _H), full(_H, 4 * _H), full(1, 4 * _H),
            full(2 * _H, 4 * _H), full(1, 4 * _H),
            full(2 * _H, _NC), full(1, _NC),
        ],
        out_specs=pl.BlockSpec((Bk, _NC), lambda b: (b, 0)),
        scratch_shapes=[
            pltpu.VMEM((T, Bk, 4 * _H), jnp.float32),
            pltpu.VMEM((T, Bk, 4 * _H), jnp.float32),
            pltpu.VMEM((T, Bk, _H), jnp.float32),
            pltpu.VMEM((T, Bk, _H), jnp.float32),
        ],
        compiler_params=pltpu.CompilerParams(
            dimension_semantics=("parallel",)),
    )(
        xt, c1w, c1s, c1t, c2w, c2s, c2t,
        l0_wih, l1_wih, l0_whh, l1_whh, l0_b, l1_b,
        l2_wih, l2_whh, l2_b, l3_wih, l3_b,
        head_w, head_b,
    )
    return out


# speedup vs baseline: 99.4360x; 1.8034x over previous
import jax
import jax.numpy as jnp
from jax.experimental import pallas as pl
from jax.experimental.pallas import tpu as pltpu

_H = 64
_NC = 8


def _sig(x):
    return 0.5 * jnp.tanh(0.5 * x) + 0.5


def _cell(g, c_prev):
    s = _sig(g[:, : 3 * _H])
    gg = jnp.tanh(g[:, 3 * _H:])
    c = s[:, _H:2 * _H] * c_prev + s[:, : _H] * gg
    return s[:, 2 * _H:] * jnp.tanh(c), c


def _conv_bn_relu(x2, bk, w_ref, s_ref, t_ref):
    n = x2.shape[0]
    row = jax.lax.broadcasted_iota(jnp.int32, x2.shape, 0)
    xm = jnp.where(row >= bk, pltpu.roll(x2, bk, 0), 0.0)
    xp = jnp.where(row < n - bk, pltpu.roll(x2, n - bk, 0), 0.0)
    acc = jnp.dot(xm, w_ref[0], preferred_element_type=jnp.float32)
    acc = acc + jnp.dot(x2, w_ref[1], preferred_element_type=jnp.float32)
    acc = acc + jnp.dot(xp, w_ref[2], preferred_element_type=jnp.float32)
    return jnp.maximum(acc * s_ref[...] + t_ref[...], 0.0)


def _fused_kernel(
    x_ref,
    c1w, c1s, c1t,
    c2w, c2s, c2t,
    w0, w1,
    l0_b, l1_b,
    w2, l2_b,
    l3_wih, l3_b,
    head_w, head_b,
    o_ref,
    h2_ref,
    hf_ref, hr_ref,
):
    T, Bk, Cin = x_ref.shape
    n = T * Bk
    zero = jnp.zeros((Bk, _H), jnp.float32)

    x2 = x_ref[...].reshape(n, Cin)
    h1 = _conv_bn_relu(x2, Bk, c1w, c1s, c1t)
    h2_ref[...] = _conv_bn_relu(h1, Bk, c2w, c2s, c2t).reshape(T, Bk, 2 * _H)

    def step1(i, carry):
        hf, cf, hr, cr = carry
        tr = T - 1 - i
        gf = jnp.dot(jnp.concatenate([h2_ref[i], hf], axis=1), w0[...],
                     preferred_element_type=jnp.float32) + l0_b[...]
        hf, cf = _cell(gf, cf)
        gr = jnp.dot(jnp.concatenate([h2_ref[tr], hr], axis=1), w1[...],
                     preferred_element_type=jnp.float32) + l1_b[...]
        hr, cr = _cell(gr, cr)
        hf_ref[i] = hf
        hr_ref[tr] = hr
        return hf, cf, hr, cr

    jax.lax.fori_loop(0, T, step1, (zero, zero, zero, zero), unroll=8)

    def step2(i, carry):
        h, c = carry
        g = jnp.dot(jnp.concatenate([hf_ref[i], hr_ref[i], h], axis=1),
                    w2[...], preferred_element_type=jnp.float32) + l2_b[...]
        return _cell(g, c)

    h2f, _ = jax.lax.fori_loop(0, T, step2, (zero, zero), unroll=8)

    g_rev = (jnp.dot(hf_ref[T - 1], l3_wih[: _H],
                     preferred_element_type=jnp.float32)
             + jnp.dot(hr_ref[T - 1], l3_wih[_H:],
                       preferred_element_type=jnp.float32)
             + l3_b[...])
    h2r, _ = _cell(g_rev, zero)

    o_ref[...] = (jnp.dot(jnp.maximum(h2f, 0.0), head_w[: _H],
                          preferred_element_type=jnp.float32)
                  + jnp.dot(jnp.maximum(h2r, 0.0), head_w[_H:],
                            preferred_element_type=jnp.float32)
                  + head_b[...])


def kernel(x, c1w, c1s, c1t, c2w, c2s, c2t,
           l0_wih, l0_whh, l0_b, l1_wih, l1_whh, l1_b,
           l2_wih, l2_whh, l2_b, l3_wih, l3_whh, l3_b,
           head_w, head_b):
    B, Cin, T = x.shape
    xt = jnp.transpose(x, (2, 0, 1))

    Bk = 128
    while B % Bk:
        Bk //= 2

    w0 = jnp.concatenate([l0_wih, l0_whh], axis=0)
    w1 = jnp.concatenate([l1_wih, l1_whh], axis=0)
    w2 = jnp.concatenate([l2_wih, l2_whh], axis=0)

    full = lambda *shape: pl.BlockSpec(shape, lambda b: (0,) * len(shape))
    out = pl.pallas_call(
        _fused_kernel,
        out_shape=jax.ShapeDtypeStruct((B, _NC), jnp.float32),
        grid=(B // Bk,),
        in_specs=[
            pl.BlockSpec((T, Bk, Cin), lambda b: (0, b, 0)),
            full(3, Cin, 64), full(1, 64), full(1, 64),
            full(3, 64, 128), full(1, 128), full(1, 128),
            full(3 * _H, 4 * _H), full(3 * _H, 4 * _H),
            full(1, 4 * _H), full(1, 4 * _H),
            full(3 * _H, 4 * _H), full(1, 4 * _H),
            full(2 * _H, 4 * _H), full(1, 4 * _H),
            full(2 * _H, _NC), full(1, _NC),
        ],
        out_specs=pl.BlockSpec((Bk, _NC), lambda b: (b, 0)),
        scratch_shapes=[
            pltpu.VMEM((T, Bk, 2 * _H), jnp.float32),
            pltpu.VMEM((T, Bk, _H), jnp.float32),
            pltpu.VMEM((T, Bk, _H), jnp.float32),
        ],
        compiler_params=pltpu.CompilerParams(
            dimension_semantics=("parallel",)),
    )(
        xt, c1w, c1s, c1t, c2w, c2s, c2t,
        w0, w1, l0_b, l1_b, w2, l2_b, l3_wih, l3_b,
        head_w, head_b,
    )
    return out
